# Initial kernel scaffold; baseline (speedup 1.0000x reference)
import jax
import jax.numpy as jnp
from jax.experimental import pallas as pl


def kernel(obs, W_gat, a_src, a_dst, W1, b1, W2, b2):
    raise NotImplementedError("write your pallas kernel here")


# speedup vs baseline: 2.0404x; 2.0404x over previous
import jax
import jax.numpy as jnp
from jax.experimental import pallas as pl
from jax.experimental.pallas import tpu as pltpu

_N = 2048
_IB = 256


def _gat_value_kernel(obs_ref, obs_t_ref, w_gat_ref, w_gat_t_ref,
                      a_src_ref, a_dst_ref, w1_ref, b1_ref, w2_ref, b2_ref,
                      out_ref):
    obs = obs_ref[0]
    obs_t = obs_t_ref[0]

    h = jnp.dot(obs, w_gat_ref[...], preferred_element_type=jnp.float32)
    ones = jnp.ones((_N, 1), dtype=jnp.float32)
    h_ext = jnp.concatenate([h, ones], axis=1)

    s_col = jnp.dot(h, a_src_ref[...], preferred_element_type=jnp.float32)
    vd_row = jnp.dot(a_dst_ref[...], w_gat_t_ref[...],
                     preferred_element_type=jnp.float32)
    d_row = jnp.dot(vd_row, obs_t, preferred_element_type=jnp.float32)

    m = jnp.max(d_row)
    p_row = jnp.exp(d_row - m)
    q_row = jnp.exp(0.2 * (d_row - m))

    sm = s_col + m
    big_m = jnp.maximum(sm, 0.2 * sm)
    u_col = jnp.exp(sm - big_m)
    v_col = jnp.exp(0.2 * sm - big_m)

    total = jnp.zeros((1, 24), dtype=jnp.float32)
    for ib in range(_N // _IB):
        sl = slice(ib * _IB, (ib + 1) * _IB)
        t = sm[sl] + (d_row - m)
        cond = t >= 0.0
        wp = jnp.where(cond, jnp.broadcast_to(p_row, (_IB, _N)), 0.0)
        wq = jnp.where(cond, 0.0, jnp.broadcast_to(q_row, (_IB, _N)))
        acc_p = jnp.dot(wp, h_ext, preferred_element_type=jnp.float32)
        acc_q = jnp.dot(wq, h_ext, preferred_element_type=jnp.float32)
        acc = u_col[sl] * acc_p + v_col[sl] * acc_q
        o = acc[:, :24] / acc[:, 24:25]
        e = jnp.where(o > 0, o, jnp.exp(o) - 1.0)
        total = total + jnp.sum(e, axis=0, keepdims=True)

    mean = total * (1.0 / _N)
    z = jnp.maximum(jnp.dot(mean, w1_ref[...],
                            preferred_element_type=jnp.float32)
                    + b1_ref[...], 0.0)
    y = jnp.dot(z, w2_ref[...], preferred_element_type=jnp.float32) + b2_ref[...]
    out_ref[0] = jnp.broadcast_to(y, (8, 128))


def kernel(obs, W_gat, a_src, a_dst, W1, b1, W2, b2):
    B = obs.shape[0]
    obs_t = jnp.swapaxes(obs, 1, 2)
    a_src_col = a_src.reshape(24, 1)
    a_dst_row = a_dst.reshape(1, 24)
    b1_row = b1.reshape(1, 36)
    b2_s = b2.reshape(1, 1)

    grid_spec = pl.GridSpec(
        grid=(B,),
        in_specs=[
            pl.BlockSpec((1, _N, obs.shape[2]), lambda b: (b, 0, 0)),
            pl.BlockSpec((1, obs.shape[2], _N), lambda b: (b, 0, 0)),
            pl.BlockSpec(W_gat.shape, lambda b: (0, 0)),
            pl.BlockSpec(W_gat.T.shape, lambda b: (0, 0)),
            pl.BlockSpec((24, 1), lambda b: (0, 0)),
            pl.BlockSpec((1, 24), lambda b: (0, 0)),
            pl.BlockSpec(W1.shape, lambda b: (0, 0)),
            pl.BlockSpec((1, 36), lambda b: (0, 0)),
            pl.BlockSpec(W2.shape, lambda b: (0, 0)),
            pl.BlockSpec((1, 1), lambda b: (0, 0)),
        ],
        out_specs=pl.BlockSpec((1, 8, 128), lambda b: (b, 0, 0)),
    )
    padded = pl.pallas_call(
        _gat_value_kernel,
        grid_spec=grid_spec,
        out_shape=jax.ShapeDtypeStruct((B, 8, 128), jnp.float32),
        compiler_params=pltpu.CompilerParams(
            dimension_semantics=("arbitrary",),
        ),
    )(obs, obs_t, W_gat, W_gat.T, a_src_col, a_dst_row, W1, b1_row, W2, b2_s)
    return padded[:, 0, :1]


# speedup vs baseline: 2.7143x; 1.3303x over previous
import jax
import jax.numpy as jnp
from jax.experimental import pallas as pl
from jax.experimental.pallas import tpu as pltpu

_N = 2048
_IB = 256


def _gat_value_kernel(obs_ref, obs_t_ref, w_gat_ref, w_gat_t_ref,
                      a_src_ref, a_dst_ref, w1_ref, b1_ref, w2_ref, b2_ref,
                      out_ref):
    obs = obs_ref[0]
    obs_t = obs_t_ref[0]

    h = jnp.dot(obs, w_gat_ref[...], preferred_element_type=jnp.float32)
    ones = jnp.ones((_N, 1), dtype=jnp.float32)
    h_ext = jnp.concatenate([h, ones], axis=1)

    s_col = jnp.dot(h, a_src_ref[...], preferred_element_type=jnp.float32)
    vd_row = jnp.dot(a_dst_ref[...], w_gat_t_ref[...],
                     preferred_element_type=jnp.float32)
    d_row = jnp.dot(vd_row, obs_t, preferred_element_type=jnp.float32)

    m = jnp.max(d_row)
    p_row = jnp.exp(d_row - m).astype(jnp.bfloat16)
    q_row = jnp.exp(0.2 * (d_row - m)).astype(jnp.bfloat16)

    sm = s_col + m
    big_m = jnp.maximum(sm, 0.2 * sm)
    u_col = jnp.exp(sm - big_m).astype(jnp.bfloat16)
    v_col = jnp.exp(0.2 * sm - big_m).astype(jnp.bfloat16)

    d_bf = (d_row - m).astype(jnp.bfloat16)
    neg_sm_bf = (-sm).astype(jnp.bfloat16)
    h_bf = h_ext.astype(jnp.bfloat16)

    total = jnp.zeros((1, 24), dtype=jnp.float32)
    for ib in range(_N // _IB):
        sl = slice(ib * _IB, (ib + 1) * _IB)
        cond = d_bf >= neg_sm_bf[sl]
        w1 = jnp.where(cond, jnp.broadcast_to(p_row, (_IB, _N)),
                       jnp.broadcast_to(q_row, (_IB, _N)))
        w2 = jnp.where(cond, u_col[sl], v_col[sl])
        w = w1 * w2
        acc = jnp.dot(w, h_bf, preferred_element_type=jnp.float32)
        o = acc[:, :24] / acc[:, 24:25]
        e = jnp.where(o > 0, o, jnp.exp(o) - 1.0)
        total = total + jnp.sum(e, axis=0, keepdims=True)

    mean = total * (1.0 / _N)
    z = jnp.maximum(jnp.dot(mean, w1_ref[...],
                            preferred_element_type=jnp.float32)
                    + b1_ref[...], 0.0)
    y = jnp.dot(z, w2_ref[...], preferred_element_type=jnp.float32) + b2_ref[...]
    out_ref[0] = jnp.broadcast_to(y, (8, 128))


def kernel(obs, W_gat, a_src, a_dst, W1, b1, W2, b2):
    B = obs.shape[0]
    obs_t = jnp.swapaxes(obs, 1, 2)
    a_src_col = a_src.reshape(24, 1)
    a_dst_row = a_dst.reshape(1, 24)
    b1_row = b1.reshape(1, 36)
    b2_s = b2.reshape(1, 1)

    grid_spec = pl.GridSpec(
        grid=(B,),
        in_specs=[
            pl.BlockSpec((1, _N, obs.shape[2]), lambda b: (b, 0, 0)),
            pl.BlockSpec((1, obs.shape[2], _N), lambda b: (b, 0, 0)),
            pl.BlockSpec(W_gat.shape, lambda b: (0, 0)),
            pl.BlockSpec(W_gat.T.shape, lambda b: (0, 0)),
            pl.BlockSpec((24, 1), lambda b: (0, 0)),
            pl.BlockSpec((1, 24), lambda b: (0, 0)),
            pl.BlockSpec(W1.shape, lambda b: (0, 0)),
            pl.BlockSpec((1, 36), lambda b: (0, 0)),
            pl.BlockSpec(W2.shape, lambda b: (0, 0)),
            pl.BlockSpec((1, 1), lambda b: (0, 0)),
        ],
        out_specs=pl.BlockSpec((1, 8, 128), lambda b: (b, 0, 0)),
    )
    padded = pl.pallas_call(
        _gat_value_kernel,
        grid_spec=grid_spec,
        out_shape=jax.ShapeDtypeStruct((B, 8, 128), jnp.float32),
        compiler_params=pltpu.CompilerParams(
            dimension_semantics=("arbitrary",),
        ),
    )(obs, obs_t, W_gat, W_gat.T, a_src_col, a_dst_row, W1, b1_row, W2, b2_s)
    return padded[:, 0, :1]
